# baseline (speedup 1.0000x reference)
import functools

import jax
import jax.numpy as jnp
from jax import lax
from jax.experimental import pallas as pl
from jax.experimental.pallas import tpu as pltpu
from jax.experimental.pallas import tpu_sc as plsc

B = 16384
D = 64
NW = 32
BPW = B // NW
CH = 128
NCH = BPW // CH
BT = 2048

def _gather4_body(uidx_hbm, iidx_hbm, t_ug, t_ig, t_um, t_im,
                  o_ug, o_ig, o_um, o_im,
                  uidx_v, iidx_v, buf0, buf1, sem0, sem1):
    wid = lax.axis_index("s") * 2 + lax.axis_index("c")
    pltpu.sync_copy(uidx_hbm.at[wid], uidx_v)
    pltpu.sync_copy(iidx_hbm.at[wid], iidx_v)
    base = wid * BPW

    steps = []
    for table, idxv, out in ((t_ug, uidx_v, o_ug), (t_ig, iidx_v, o_ig),
                             (t_um, uidx_v, o_um), (t_im, iidx_v, o_im)):
        for j in range(NCH):
            steps.append((table, idxv, j, out, base + j * CH))

    bufs = (buf0, buf1)
    sems = (sem0, sem1)
    table, idxv, j, out, off = steps[0]
    prev = pltpu.async_copy(table.at[idxv.at[j]], bufs[0], sems[0])
    for k in range(1, len(steps)):
        table, idxv, j, out, off = steps[k]
        cur = pltpu.async_copy(table.at[idxv.at[j]], bufs[k % 2], sems[k % 2])
        prev.wait()
        ptable, pidxv, pj, pout, poff = steps[k - 1]
        pltpu.sync_copy(bufs[(k - 1) % 2], pout.at[pl.ds(poff, CH)])
        prev = cur
    prev.wait()
    table, idxv, j, out, off = steps[-1]
    pltpu.sync_copy(bufs[(len(steps) - 1) % 2], out.at[pl.ds(off, CH)])


@functools.lru_cache(maxsize=1)
def _get_gather4():
    mesh = plsc.VectorSubcoreMesh(core_axis_name="c", subcore_axis_name="s")
    return pl.kernel(
        _gather4_body,
        mesh=mesh,
        out_type=[jax.ShapeDtypeStruct((B, D), jnp.float32) for _ in range(4)],
        scratch_types=[
            pltpu.VMEM((NCH, CH), jnp.int32),
            pltpu.VMEM((NCH, CH), jnp.int32),
            pltpu.VMEM((CH, D), jnp.float32),
            pltpu.VMEM((CH, D), jnp.float32),
            pltpu.SemaphoreType.DMA,
            pltpu.SemaphoreType.DMA,
        ],
        compiler_params=pltpu.CompilerParams(use_tc_tiling_on_sc=False),
    )


def _fuse_body(ug, ig, um, im, w1a, w1b, b1, wg, wm, bo, out):
    h = jnp.dot(um[...], w1a[...], preferred_element_type=jnp.float32)
    h = h + jnp.dot(im[...], w1b[...], preferred_element_type=jnp.float32)
    h = jnp.maximum(h + b1[...], 0.0)
    g = ug[...] * ig[...]
    out[...] = (jnp.dot(g, wg[...], preferred_element_type=jnp.float32)
                + jnp.dot(h, wm[...], preferred_element_type=jnp.float32)
                + bo[...])


_fuse = pl.pallas_call(
    _fuse_body,
    grid=(B // BT,),
    in_specs=[
        pl.BlockSpec((BT, D), lambda i: (i, 0)),
        pl.BlockSpec((BT, D), lambda i: (i, 0)),
        pl.BlockSpec((BT, D), lambda i: (i, 0)),
        pl.BlockSpec((BT, D), lambda i: (i, 0)),
        pl.BlockSpec((D, D), lambda i: (0, 0)),
        pl.BlockSpec((D, D), lambda i: (0, 0)),
        pl.BlockSpec((1, D), lambda i: (0, 0)),
        pl.BlockSpec((D, 1), lambda i: (0, 0)),
        pl.BlockSpec((D, 1), lambda i: (0, 0)),
        pl.BlockSpec((1, 1), lambda i: (0, 0)),
    ],
    out_specs=pl.BlockSpec((BT, 1), lambda i: (i, 0)),
    out_shape=jax.ShapeDtypeStruct((B, 1), jnp.float32),
)


def kernel(user_indices, item_indices, user_emb_ncf, item_emb_ncf,
           user_emb_mlp, item_emb_mlp, W1, b1, W_out, b_out):
    uidx = user_indices.astype(jnp.int32).reshape(NW, NCH, CH)
    iidx = item_indices.astype(jnp.int32).reshape(NW, NCH, CH)
    ug, ig, um, im = _get_gather4()(uidx, iidx, user_emb_ncf, item_emb_ncf,
                                    user_emb_mlp, item_emb_mlp)
    return _fuse(ug, ig, um, im, W1[:D], W1[D:], b1.reshape(1, D),
                 W_out[:D], W_out[D:], b_out.reshape(1, 1))


# speedup vs baseline: 1.0027x; 1.0027x over previous
import functools

import jax
import jax.numpy as jnp
from jax import lax
from jax.experimental import pallas as pl
from jax.experimental.pallas import tpu as pltpu
from jax.experimental.pallas import tpu_sc as plsc

B = 16384
D = 64
NW = 32
BPW = B // NW
CH = 128
NCH = BPW // CH
BT = 2048


def _gather1_body(idx_hbm, table, out, idx_v, buf0, buf1, sem0, sem1):
    wid = lax.axis_index("s") * 2 + lax.axis_index("c")
    pltpu.sync_copy(idx_hbm.at[wid], idx_v)
    base = wid * BPW

    bufs = (buf0, buf1)
    sems = (sem0, sem1)
    prev = pltpu.async_copy(table.at[idx_v.at[0]], bufs[0], sems[0])
    for j in range(1, NCH):
        cur = pltpu.async_copy(table.at[idx_v.at[j]], bufs[j % 2], sems[j % 2])
        prev.wait()
        pltpu.sync_copy(bufs[(j - 1) % 2], out.at[pl.ds(base + (j - 1) * CH, CH)])
        prev = cur
    prev.wait()
    pltpu.sync_copy(bufs[(NCH - 1) % 2], out.at[pl.ds(base + (NCH - 1) * CH, CH)])


@functools.lru_cache(maxsize=1)
def _get_gather1():
    mesh = plsc.VectorSubcoreMesh(core_axis_name="c", subcore_axis_name="s")
    return pl.kernel(
        _gather1_body,
        mesh=mesh,
        out_type=jax.ShapeDtypeStruct((B, D), jnp.float32),
        scratch_types=[
            pltpu.VMEM((NCH, CH), jnp.int32),
            pltpu.VMEM((CH, D), jnp.float32),
            pltpu.VMEM((CH, D), jnp.float32),
            pltpu.SemaphoreType.DMA,
            pltpu.SemaphoreType.DMA,
        ],
        compiler_params=pltpu.CompilerParams(use_tc_tiling_on_sc=False),
    )


def _fuse_body(ug, ig, um, im, w1a, w1b, b1, wg, wm, bo, out):
    h = jnp.dot(um[...], w1a[...], preferred_element_type=jnp.float32)
    h = h + jnp.dot(im[...], w1b[...], preferred_element_type=jnp.float32)
    h = jnp.maximum(h + b1[...], 0.0)
    g = ug[...] * ig[...]
    out[...] = (jnp.dot(g, wg[...], preferred_element_type=jnp.float32)
                + jnp.dot(h, wm[...], preferred_element_type=jnp.float32)
                + bo[...])


_fuse = pl.pallas_call(
    _fuse_body,
    grid=(B // BT,),
    in_specs=[
        pl.BlockSpec((BT, D), lambda i: (i, 0)),
        pl.BlockSpec((BT, D), lambda i: (i, 0)),
        pl.BlockSpec((BT, D), lambda i: (i, 0)),
        pl.BlockSpec((BT, D), lambda i: (i, 0)),
        pl.BlockSpec((D, D), lambda i: (0, 0)),
        pl.BlockSpec((D, D), lambda i: (0, 0)),
        pl.BlockSpec((1, D), lambda i: (0, 0)),
        pl.BlockSpec((D, 1), lambda i: (0, 0)),
        pl.BlockSpec((D, 1), lambda i: (0, 0)),
        pl.BlockSpec((1, 1), lambda i: (0, 0)),
    ],
    out_specs=pl.BlockSpec((BT, 1), lambda i: (i, 0)),
    out_shape=jax.ShapeDtypeStruct((B, 1), jnp.float32),
)


def kernel(user_indices, item_indices, user_emb_ncf, item_emb_ncf,
           user_emb_mlp, item_emb_mlp, W1, b1, W_out, b_out):
    uidx = user_indices.astype(jnp.int32).reshape(NW, NCH, CH)
    iidx = item_indices.astype(jnp.int32).reshape(NW, NCH, CH)
    g1 = _get_gather1()
    ug = g1(uidx, user_emb_ncf)
    ig = g1(iidx, item_emb_ncf)
    um = g1(uidx, user_emb_mlp)
    im = g1(iidx, item_emb_mlp)
    return _fuse(ug, ig, um, im, W1[:D], W1[D:], b1.reshape(1, D),
                 W_out[:D], W_out[D:], b_out.reshape(1, 1))
